# Initial kernel scaffold; baseline (speedup 1.0000x reference)
import jax
import jax.numpy as jnp
from jax.experimental import pallas as pl


def kernel(premise_indices, hypothesis_indices, table_prem, table_hypo, W, b):
    raise NotImplementedError("write your pallas kernel here")


# speedup vs baseline: 2.4371x; 2.4371x over previous
import functools

import jax
import jax.numpy as jnp
from jax import lax
from jax.experimental import pallas as pl
from jax.experimental.pallas import tpu as pltpu
from jax.experimental.pallas import tpu_sc as plsc

_B = 16384
_L = 50
_EMB = 64
_NW = 32
_SPW = _B // _NW
_LANES = 16
_G = _EMB // _LANES

_mesh = plsc.VectorSubcoreMesh(core_axis_name="c", subcore_axis_name="s")


@functools.partial(
    pl.kernel,
    mesh=_mesh,
    out_type=jax.ShapeDtypeStruct((_B, 2 * _EMB), jnp.float32),
    scratch_types=[
        pltpu.VMEM((_SPW, _L), jnp.int32),
        pltpu.VMEM((_L, _EMB), jnp.float32),
        pltpu.VMEM((_L, _EMB), jnp.float32),
        pltpu.VMEM((_SPW, 2 * _EMB), jnp.float32),
        pltpu.SemaphoreType.DMA,
        pltpu.SemaphoreType.DMA,
    ],
    compiler_params=pltpu.CompilerParams(use_tc_tiling_on_sc=False),
)
def _pool_kernel(prem_idx, hypo_idx, tab_p, tab_h, out, idx_v, rows0, rows1,
                 pooled, sem0, sem1):
    wid = lax.axis_index("s") * 2 + lax.axis_index("c")
    base = wid * _SPW

    def accumulate(rows_ref, sample, col):
        def body(r, carry):
            return tuple(
                carry[g] + rows_ref[r, pl.ds(g * _LANES, _LANES)]
                for g in range(_G)
            )
        z = jnp.zeros((_LANES,), jnp.float32)
        acc = lax.fori_loop(0, _L, body, (z,) * _G)
        for g in range(_G):
            pooled[sample, pl.ds(col + g * _LANES, _LANES)] = acc[g]

    for half, (idx_hbm, tab) in enumerate(((prem_idx, tab_p), (hypo_idx, tab_h))):
        col = half * _EMB
        pltpu.sync_copy(idx_hbm.at[pl.ds(base, _SPW)], idx_v)

        pltpu.async_copy(tab.at[idx_v.at[0]], rows0, sem0)
        pltpu.async_copy(tab.at[idx_v.at[1]], rows1, sem1)

        def step(j, _):
            s0 = 2 * j
            pltpu.make_async_copy(tab.at[idx_v.at[s0]], rows0, sem0).wait()
            accumulate(rows0, s0, col)

            @pl.when(j < _SPW // 2 - 1)
            def _():
                pltpu.async_copy(tab.at[idx_v.at[s0 + 2]], rows0, sem0)

            pltpu.make_async_copy(tab.at[idx_v.at[s0 + 1]], rows1, sem1).wait()
            accumulate(rows1, s0 + 1, col)

            @pl.when(j < _SPW // 2 - 1)
            def _():
                pltpu.async_copy(tab.at[idx_v.at[s0 + 3]], rows1, sem1)

            return 0

        lax.fori_loop(0, _SPW // 2, step, 0)

    pltpu.sync_copy(pooled, out.at[pl.ds(base, _SPW)])


def _linear_body(x_ref, w_ref, b_ref, o_ref):
    o_ref[...] = (
        jnp.dot(x_ref[...], w_ref[...], preferred_element_type=jnp.float32)
        + b_ref[...]
    )


def kernel(premise_indices, hypothesis_indices, table_prem, table_hypo, W, b):
    pooled = _pool_kernel(
        premise_indices.astype(jnp.int32),
        hypothesis_indices.astype(jnp.int32),
        table_prem,
        table_hypo,
    )
    w_scaled = W * (1.0 / _L)
    b2 = b.reshape(1, 3)
    bm = 2048
    probs = pl.pallas_call(
        _linear_body,
        grid=(_B // bm,),
        in_specs=[
            pl.BlockSpec((bm, 2 * _EMB), lambda i: (i, 0)),
            pl.BlockSpec((2 * _EMB, 3), lambda i: (0, 0)),
            pl.BlockSpec((1, 3), lambda i: (0, 0)),
        ],
        out_specs=pl.BlockSpec((bm, 3), lambda i: (i, 0)),
        out_shape=jax.ShapeDtypeStruct((_B, 3), jnp.float32),
    )(pooled, w_scaled, b2)
    return probs


# speedup vs baseline: 2.5836x; 1.0601x over previous
import functools

import jax
import jax.numpy as jnp
from jax import lax
from jax.experimental import pallas as pl
from jax.experimental.pallas import tpu as pltpu
from jax.experimental.pallas import tpu_sc as plsc

_B = 16384
_L = 50
_EMB = 64
_NW = 32
_SPW = _B // _NW
_SEG = 256
_NSEG = _SPW // _SEG
_NBUF = 4
_LANES = 16
_G = _EMB // _LANES
_UNROLL = 10

_mesh = plsc.VectorSubcoreMesh(core_axis_name="c", subcore_axis_name="s")


@functools.partial(
    pl.kernel,
    mesh=_mesh,
    out_type=jax.ShapeDtypeStruct((_B, 2 * _EMB), jnp.float32),
    scratch_types=[
        pltpu.VMEM((_SEG, _L), jnp.int32),
        pltpu.VMEM((_SEG, _EMB), jnp.int32),
        pltpu.VMEM((_NBUF, _L, 2 * _EMB), jnp.float32),
        pltpu.VMEM((_SPW, 2 * _EMB), jnp.float32),
        pltpu.SemaphoreType.DMA,
        pltpu.SemaphoreType.DMA,
        pltpu.SemaphoreType.DMA,
        pltpu.SemaphoreType.DMA,
    ],
    compiler_params=pltpu.CompilerParams(use_tc_tiling_on_sc=False),
)
def _pool_kernel(sup_p, cb_p, sup_h, cb_h, tab_p, tab_h, out,
                 sup_v, cb_v, rows, pooled, sem0, sem1, sem2, sem3):
    sems = (sem0, sem1, sem2, sem3)
    wid = lax.axis_index("s") * 2 + lax.axis_index("c")
    base = wid * _SPW

    def accumulate(rows_ref, smp, dst_row, col):
        def rbody(r0, carry):
            accs = carry
            for k in range(_UNROLL):
                r = r0 * _UNROLL + k
                cbs = cb_v[smp, pl.ds(r, _LANES)][0]
                accs = tuple(
                    accs[g] + rows_ref[r, pl.ds(cbs + g * _LANES, _LANES)]
                    for g in range(_G)
                )
            return accs
        z = jnp.zeros((_LANES,), jnp.float32)
        acc = lax.fori_loop(0, _L // _UNROLL, rbody, (z,) * _G)
        for g in range(_G):
            pooled[dst_row, pl.ds(col + g * _LANES, _LANES)] = acc[g]

    for half, (sup_hbm, cb_hbm, tab) in enumerate(
            ((sup_p, cb_p, tab_p), (sup_h, cb_h, tab_h))):
        col = half * _EMB

        def seg_body(seg, _):
            s0 = base + seg * _SEG
            pltpu.sync_copy(sup_hbm.at[pl.ds(s0, _SEG)], sup_v)
            pltpu.sync_copy(cb_hbm.at[pl.ds(s0, _SEG)], cb_v)
            for b in range(_NBUF):
                pltpu.async_copy(tab.at[sup_v.at[b]], rows.at[b], sems[b])

            def j_body(j, _):
                for b in range(_NBUF):
                    smp = _NBUF * j + b
                    pltpu.make_async_copy(
                        tab.at[sup_v.at[smp]], rows.at[b], sems[b]).wait()
                    accumulate(rows.at[b], smp, seg * _SEG + smp, col)

                    @pl.when(smp + _NBUF < _SEG)
                    def _():
                        pltpu.async_copy(
                            tab.at[sup_v.at[smp + _NBUF]], rows.at[b], sems[b])
                return 0

            lax.fori_loop(0, _SEG // _NBUF, j_body, 0)
            return 0

        lax.fori_loop(0, _NSEG, seg_body, 0)

    pltpu.sync_copy(pooled, out.at[pl.ds(base, _SPW)])


def _linear_body(x_ref, w_ref, b_ref, o_ref):
    o_ref[...] = (
        jnp.dot(x_ref[...], w_ref[...], preferred_element_type=jnp.float32)
        + b_ref[...]
    )


def kernel(premise_indices, hypothesis_indices, table_prem, table_hypo, W, b):
    pi = premise_indices.astype(jnp.int32)
    hi = hypothesis_indices.astype(jnp.int32)
    pad = jnp.zeros((_B, _EMB - _L), jnp.int32)
    sup_p = pi >> 1
    cb_p = jnp.concatenate([(pi & 1) << 6, pad], axis=1)
    sup_h = hi >> 1
    cb_h = jnp.concatenate([(hi & 1) << 6, pad], axis=1)
    tab_p = table_prem.reshape(500000, 2 * _EMB)
    tab_h = table_hypo.reshape(500000, 2 * _EMB)

    pooled = _pool_kernel(sup_p, cb_p, sup_h, cb_h, tab_p, tab_h)

    w_scaled = W * (1.0 / _L)
    b2 = b.reshape(1, 3)
    bm = 2048
    probs = pl.pallas_call(
        _linear_body,
        grid=(_B // bm,),
        in_specs=[
            pl.BlockSpec((bm, 2 * _EMB), lambda i: (i, 0)),
            pl.BlockSpec((2 * _EMB, 3), lambda i: (0, 0)),
            pl.BlockSpec((1, 3), lambda i: (0, 0)),
        ],
        out_specs=pl.BlockSpec((bm, 3), lambda i: (i, 0)),
        out_shape=jax.ShapeDtypeStruct((_B, 3), jnp.float32),
    )(pooled, w_scaled, b2)
    return probs


# speedup vs baseline: 2.8374x; 1.0983x over previous
import functools

import jax
import jax.numpy as jnp
from jax import lax
from jax.experimental import pallas as pl
from jax.experimental.pallas import tpu as pltpu
from jax.experimental.pallas import tpu_sc as plsc

_B = 16384
_L = 50
_EMB = 64
_NW = 32
_SPW = _B // _NW
_SEG = 256
_NSEG = _SPW // _SEG
_NBUF = 4
_LANES = 16
_G = _EMB // _LANES
_UNROLL = 10

_mesh = plsc.VectorSubcoreMesh(core_axis_name="c", subcore_axis_name="s")


@functools.partial(
    pl.kernel,
    mesh=_mesh,
    out_type=jax.ShapeDtypeStruct((_B, 2 * _EMB), jnp.float32),
    scratch_types=[
        pltpu.VMEM((_SEG, _L), jnp.int32),
        pltpu.VMEM((_NBUF, _L, _EMB), jnp.float32),
        pltpu.VMEM((_SPW, 2 * _EMB), jnp.float32),
        pltpu.SemaphoreType.DMA,
        pltpu.SemaphoreType.DMA,
        pltpu.SemaphoreType.DMA,
        pltpu.SemaphoreType.DMA,
    ],
    compiler_params=pltpu.CompilerParams(use_tc_tiling_on_sc=False),
)
def _pool_kernel(idx_p, idx_h, tab_p, tab_h, out,
                 idx_v, rows, pooled, sem0, sem1, sem2, sem3):
    sems = (sem0, sem1, sem2, sem3)
    wid = lax.axis_index("s") * 2 + lax.axis_index("c")
    base = wid * _SPW

    def accumulate(rows_ref, dst_row, col):
        def rbody(r0, carry):
            accs = carry
            for k in range(_UNROLL):
                r = r0 * _UNROLL + k
                accs = tuple(
                    accs[g] + rows_ref[r, pl.ds(g * _LANES, _LANES)]
                    for g in range(_G)
                )
            return accs
        z = jnp.zeros((_LANES,), jnp.float32)
        acc = lax.fori_loop(0, _L // _UNROLL, rbody, (z,) * _G)
        for g in range(_G):
            pooled[dst_row, pl.ds(col + g * _LANES, _LANES)] = acc[g]

    for half, (idx_hbm, tab) in enumerate(((idx_p, tab_p), (idx_h, tab_h))):
        col = half * _EMB

        def seg_body(seg, _):
            s0 = base + seg * _SEG
            pltpu.sync_copy(idx_hbm.at[pl.ds(s0, _SEG)], idx_v)
            for b in range(_NBUF):
                pltpu.async_copy(tab.at[idx_v.at[b]], rows.at[b], sems[b])

            def j_body(j, _):
                for b in range(_NBUF):
                    smp = _NBUF * j + b
                    pltpu.make_async_copy(
                        tab.at[idx_v.at[smp]], rows.at[b], sems[b]).wait()
                    accumulate(rows.at[b], seg * _SEG + smp, col)

                    @pl.when(smp + _NBUF < _SEG)
                    def _():
                        pltpu.async_copy(
                            tab.at[idx_v.at[smp + _NBUF]], rows.at[b], sems[b])
                return 0

            lax.fori_loop(0, _SEG // _NBUF, j_body, 0)
            return 0

        lax.fori_loop(0, _NSEG, seg_body, 0)

    pltpu.sync_copy(pooled, out.at[pl.ds(base, _SPW)])


def _linear_body(x_ref, w_ref, b_ref, o_ref):
    o_ref[...] = (
        jnp.dot(x_ref[...], w_ref[...], preferred_element_type=jnp.float32)
        + b_ref[...]
    )


def kernel(premise_indices, hypothesis_indices, table_prem, table_hypo, W, b):
    pi = premise_indices.astype(jnp.int32)
    hi = hypothesis_indices.astype(jnp.int32)

    pooled = _pool_kernel(pi, hi, table_prem, table_hypo)

    w_scaled = W * (1.0 / _L)
    b2 = b.reshape(1, 3)
    bm = 2048
    probs = pl.pallas_call(
        _linear_body,
        grid=(_B // bm,),
        in_specs=[
            pl.BlockSpec((bm, 2 * _EMB), lambda i: (i, 0)),
            pl.BlockSpec((2 * _EMB, 3), lambda i: (0, 0)),
            pl.BlockSpec((1, 3), lambda i: (0, 0)),
        ],
        out_specs=pl.BlockSpec((bm, 3), lambda i: (i, 0)),
        out_shape=jax.ShapeDtypeStruct((_B, 3), jnp.float32),
    )(pooled, w_scaled, b2)
    return probs
